# baseline (speedup 1.0000x reference)
import functools

import jax
import jax.numpy as jnp
from jax import lax
from jax.experimental import pallas as pl
from jax.experimental.pallas import tpu as pltpu
from jax.experimental.pallas import tpu_sc as plsc

_T = 1000
_BATCH = 16384
_NC = 1
_NS = 16
_NW = _NC * _NS
_BPW = _BATCH // _NW
_L = 16
_CHUNKS = _BPW // _L


def _make_gather():
    mesh = plsc.VectorSubcoreMesh(core_axis_name="c", subcore_axis_name="s",
                                  num_cores=_NC)

    @functools.partial(
        pl.kernel,
        mesh=mesh,
        out_type=jax.ShapeDtypeStruct((_BATCH,), jnp.float32),
        scratch_types=[
            pltpu.VMEM((_T,), jnp.float32),
            pltpu.VMEM((_BPW,), jnp.int32),
            pltpu.VMEM((_BPW,), jnp.float32),
            pltpu.SemaphoreType.DMA,
        ],
        compiler_params=pltpu.CompilerParams(needs_layout_passes=False),
    )
    def gather_kernel(table_hbm, t_hbm, out_hbm, table_v, idx_v, res_v,
                      sem):
        wid = lax.axis_index("s") * _NC + lax.axis_index("c")
        base = wid * _BPW
        cp_tab = pltpu.async_copy(table_hbm, table_v, sem)
        cp_idx = pltpu.async_copy(t_hbm.at[pl.ds(base, _BPW)], idx_v, sem)
        cp_tab.wait()
        cp_idx.wait()
        @plsc.parallel_loop(0, _BPW, _L, unroll=8)
        def _body(off):
            idx = idx_v[pl.ds(off, _L)]
            res_v[pl.ds(off, _L)] = plsc.load_gather(table_v, [idx])
        pltpu.sync_copy(res_v, out_hbm.at[pl.ds(base, _BPW)])

    return gather_kernel


_gather = _make_gather()


def kernel(sqrt_alphas_cumprod, t):
    out = _gather(sqrt_alphas_cumprod, t)
    return out.reshape(-1, 1, 1)


# speedup vs baseline: 1.0025x; 1.0025x over previous
import functools

import jax
import jax.numpy as jnp
from jax import lax
from jax.experimental import pallas as pl
from jax.experimental.pallas import tpu as pltpu
from jax.experimental.pallas import tpu_sc as plsc

_T = 1000
_BATCH = 16384
_NC = 1
_NS = 16
_NW = _NC * _NS
_BPW = _BATCH // _NW
_L = 16


def _make_gather():
    mesh = plsc.VectorSubcoreMesh(core_axis_name="c", subcore_axis_name="s",
                                  num_cores=_NC)

    @functools.partial(
        pl.kernel,
        mesh=mesh,
        out_type=jax.ShapeDtypeStruct((_BATCH,), jnp.float32),
        scratch_types=[
            pltpu.VMEM((_T,), jnp.float32),
            pltpu.VMEM((_BPW,), jnp.int32),
            pltpu.VMEM((_BPW,), jnp.float32),
            pltpu.SemaphoreType.DMA,
        ],
        compiler_params=pltpu.CompilerParams(needs_layout_passes=False),
    )
    def gather_kernel(table_hbm, t_hbm, out_hbm, table_v, idx_v, res_v,
                      sem):
        wid = lax.axis_index("s") * _NC + lax.axis_index("c")
        base = wid * _BPW
        cp_tab = pltpu.async_copy(table_hbm, table_v, sem)
        cp_idx = pltpu.async_copy(t_hbm.at[pl.ds(base, _BPW)], idx_v, sem)
        cp_tab.wait()
        cp_idx.wait()

        @plsc.parallel_loop(0, _BPW, _L, unroll=8)
        def _body(off):
            idx = idx_v[pl.ds(off, _L)]
            res_v[pl.ds(off, _L)] = plsc.load_gather(table_v, [idx])

        pltpu.sync_copy(res_v, out_hbm.at[pl.ds(base, _BPW)])

    return gather_kernel


_gather = _make_gather()


def kernel(sqrt_alphas_cumprod, t):
    out = _gather(sqrt_alphas_cumprod, t)
    return out.reshape(-1, 1, 1)
